# baseline (speedup 1.0000x reference)
import functools

import jax
import jax.numpy as jnp
from jax import lax
from jax.experimental import pallas as pl
from jax.experimental.pallas import tpu as pltpu
from jax.experimental.pallas import tpu_sc as plsc

B, S, NSW = 1024, 50, 20
EMB = 64
T = B * S
NC, NS = 2, 16
NW = NC * NS
TPW = T // NW
IDX_COLS = 128
CHUNK_TOK = 32
CHUNK_IDX_ROWS = CHUNK_TOK * NSW // IDX_COLS
NCHUNK = TPW // CHUNK_TOK
ROWS_PER_CHUNK = CHUNK_TOK * NSW

VOCAB_P = 1000064
VBLK = 128
NBLK = VOCAB_P // VBLK
BLK_PER_W = NBLK // NW
NBLK_TAIL = NBLK - BLK_PER_W * NW


def _transpose_kernel(tt_hbm, flat_hbm, buf0, buf1, ob0, ob1, sem0, sem1):
    wid = lax.axis_index("s") * NC + lax.axis_index("c")

    def fire(c, buf, sem):
        pltpu.async_copy(tt_hbm.at[:, pl.ds(c * VBLK, VBLK)], buf, sem)

    def drain(buf, sem):
        pltpu.make_async_copy(tt_hbm.at[:, pl.ds(0, VBLK)], buf, sem).wait()

    lane = lax.iota(jnp.int32, 16)

    def compute(c, buf, ob):
        @pl.loop(0, VBLK)
        def _(v):
            vcol = lax.broadcast(v, (16,))
            for g in range(EMB // 16):
                vals = plsc.load_gather(buf, [lane + (16 * g), vcol])
                ob[pl.ds(v * EMB + 16 * g, 16)] = vals
        pltpu.sync_copy(ob, flat_hbm.at[pl.ds(c * (VBLK * EMB), VBLK * EMB)])

    c0 = wid * BLK_PER_W
    fire(c0, buf0, sem0)

    @pl.loop(0, BLK_PER_W, step=2)
    def _(k):
        fire(c0 + k + 1, buf1, sem1)
        drain(buf0, sem0)
        compute(c0 + k, buf0, ob0)

        @pl.when(k + 2 < BLK_PER_W)
        def _():
            fire(c0 + k + 2, buf0, sem0)

        drain(buf1, sem1)
        compute(c0 + k + 1, buf1, ob1)

    @pl.when(wid < NBLK_TAIL)
    def _():
        c_tail = NW * BLK_PER_W + wid
        fire(c_tail, buf0, sem0)
        drain(buf0, sem0)
        compute(c_tail, buf0, ob0)


def _sc_kernel(table_hbm, idx_hbm, out_hbm,
               idx_v0, idx_v1, rows_v0, rows_v1, out_v, sem0, sem1):
    wid = lax.axis_index("s") * NC + lax.axis_index("c")
    tok_base = wid * TPW
    idx_base = wid * (TPW * NSW)

    def fire(g, idx_v, rows_v, sem):
        i0 = idx_base + g * ROWS_PER_CHUNK
        pltpu.sync_copy(idx_hbm.at[pl.ds(i0, ROWS_PER_CHUNK)], idx_v)
        for j in range(CHUNK_IDX_ROWS):
            pltpu.async_copy(
                table_hbm.at[idx_v.at[pl.ds(j * IDX_COLS, IDX_COLS)]],
                rows_v.at[pl.ds(j * IDX_COLS, IDX_COLS)],
                sem,
            )

    def drain(rows_v, sem):
        pltpu.make_async_copy(
            table_hbm.at[pl.ds(0, ROWS_PER_CHUNK)], rows_v, sem
        ).wait()

    def compute(g, rows_v):
        @pl.loop(0, CHUNK_TOK)
        def _(t):
            r0 = t * NSW
            for c in range(EMB // 16):
                lanes = pl.ds(c * 16, 16)
                acc = rows_v[r0, lanes]
                for s in range(1, NSW):
                    acc = acc + rows_v[r0 + s, lanes]
                out_v[t, lanes] = acc * (1.0 / NSW)
        pltpu.sync_copy(
            out_v, out_hbm.at[pl.ds(tok_base + g * CHUNK_TOK, CHUNK_TOK)]
        )

    fire(0, idx_v0, rows_v0, sem0)

    @pl.loop(0, NCHUNK, step=2)
    def _(g):
        fire(g + 1, idx_v1, rows_v1, sem1)
        drain(rows_v0, sem0)
        compute(g, rows_v0)

        @pl.when(g + 2 < NCHUNK)
        def _():
            fire(g + 2, idx_v0, rows_v0, sem0)

        drain(rows_v1, sem1)
        compute(g + 1, rows_v1)


@jax.jit
def _pooled_lookup(table, idx_flat):
    mesh = plsc.VectorSubcoreMesh(core_axis_name="c", subcore_axis_name="s")

    detranspose = pl.kernel(
        _transpose_kernel,
        out_type=jax.ShapeDtypeStruct((VOCAB_P * EMB,), jnp.float32),
        mesh=mesh,
        compiler_params=pltpu.CompilerParams(
            use_tc_tiling_on_sc=True, needs_layout_passes=False
        ),
        scratch_types=[
            pltpu.VMEM((EMB, VBLK), jnp.float32),
            pltpu.VMEM((EMB, VBLK), jnp.float32),
            pltpu.VMEM((VBLK * EMB,), jnp.float32),
            pltpu.VMEM((VBLK * EMB,), jnp.float32),
            pltpu.SemaphoreType.DMA,
            pltpu.SemaphoreType.DMA,
        ],
    )
    flat = detranspose(table.T)
    table_lin = flat.reshape(VOCAB_P, EMB)

    run = pl.kernel(
        _sc_kernel,
        out_type=jax.ShapeDtypeStruct((T, EMB), jnp.float32),
        mesh=mesh,
        compiler_params=pltpu.CompilerParams(use_tc_tiling_on_sc=False),
        scratch_types=[
            pltpu.VMEM((ROWS_PER_CHUNK,), jnp.int32),
            pltpu.VMEM((ROWS_PER_CHUNK,), jnp.int32),
            pltpu.VMEM((ROWS_PER_CHUNK, EMB), jnp.float32),
            pltpu.VMEM((ROWS_PER_CHUNK, EMB), jnp.float32),
            pltpu.VMEM((CHUNK_TOK, EMB), jnp.float32),
            pltpu.SemaphoreType.DMA,
            pltpu.SemaphoreType.DMA,
        ],
    )
    return run(table_lin, idx_flat)


def kernel(inpt, table):
    idx_flat = inpt.reshape(T * NSW)
    out = _pooled_lookup(table, idx_flat)
    return out.reshape(B, S, EMB)


# speedup vs baseline: 2.2112x; 2.2112x over previous
import functools

import jax
import jax.numpy as jnp
from jax import lax
from jax.experimental import pallas as pl
from jax.experimental.pallas import tpu as pltpu
from jax.experimental.pallas import tpu_sc as plsc

B, S, NSW = 1024, 50, 20
EMB = 64
ROWW = 128
VPAD = 1000008
T = B * S
NC, NS = 2, 16
NW = NC * NS
TPW = T // NW
CHUNK_TOK = 20
ROWS_PER_CHUNK = CHUNK_TOK * NSW
GATHER_N = 5
GATHER_IDX = ROWS_PER_CHUNK // GATHER_N
NCHUNK = TPW // CHUNK_TOK


def _sc_kernel(table_hbm, idx_hbm, out_hbm,
               idx_v0, idx_v1, rows_v0, rows_v1, out_v, sem0, sem1):
    wid = lax.axis_index("s") * NC + lax.axis_index("c")
    out_base = wid * (TPW * EMB)
    idx_base = wid * (TPW * NSW)

    def fire(g, idx_v, rows_v, sem):
        i0 = idx_base + g * ROWS_PER_CHUNK
        pltpu.sync_copy(idx_hbm.at[pl.ds(i0, ROWS_PER_CHUNK)], idx_v)
        for j in range(GATHER_N):
            pltpu.async_copy(
                table_hbm.at[idx_v.at[pl.ds(j * GATHER_IDX, GATHER_IDX)]],
                rows_v.at[pl.ds(j * GATHER_IDX, GATHER_IDX)],
                sem,
            )

    def drain(rows_v, sem):
        pltpu.make_async_copy(
            table_hbm.at[pl.ds(0, ROWS_PER_CHUNK)], rows_v, sem
        ).wait()

    def compute(g, rows_v):
        @pl.loop(0, CHUNK_TOK)
        def _(t):
            r0 = t * NSW
            for c in range(EMB // 16):
                lanes = pl.ds(c * 16, 16)
                acc = rows_v[r0, lanes]
                for s in range(1, NSW):
                    acc = acc + rows_v[r0 + s, lanes]
                out_v[pl.ds(t * EMB + c * 16, 16)] = acc * (1.0 / NSW)
        pltpu.sync_copy(
            out_v,
            out_hbm.at[pl.ds(out_base + g * (CHUNK_TOK * EMB), CHUNK_TOK * EMB)],
        )

    fire(0, idx_v0, rows_v0, sem0)

    @pl.loop(0, NCHUNK, step=2)
    def _(g):
        fire(g + 1, idx_v1, rows_v1, sem1)
        drain(rows_v0, sem0)
        compute(g, rows_v0)

        @pl.when(g + 2 < NCHUNK)
        def _():
            fire(g + 2, idx_v0, rows_v0, sem0)

        drain(rows_v1, sem1)
        compute(g + 1, rows_v1)


@jax.jit
def _pooled_lookup(table, idx_flat):
    mesh = plsc.VectorSubcoreMesh(core_axis_name="c", subcore_axis_name="s")
    run = pl.kernel(
        _sc_kernel,
        out_type=jax.ShapeDtypeStruct((T * EMB,), jnp.float32),
        mesh=mesh,
        compiler_params=pltpu.CompilerParams(use_tc_tiling_on_sc=True),
        scratch_types=[
            pltpu.VMEM((ROWS_PER_CHUNK,), jnp.int32),
            pltpu.VMEM((ROWS_PER_CHUNK,), jnp.int32),
            pltpu.VMEM((ROWS_PER_CHUNK, ROWW), jnp.float32),
            pltpu.VMEM((ROWS_PER_CHUNK, ROWW), jnp.float32),
            pltpu.VMEM((CHUNK_TOK * EMB,), jnp.float32),
            pltpu.SemaphoreType.DMA,
            pltpu.SemaphoreType.DMA,
        ],
    )
    tablep = jnp.pad(table, ((0, VPAD - table.shape[0]), (0, ROWW - EMB)))
    return run(tablep, idx_flat)


def kernel(inpt, table):
    idx_flat = inpt.reshape(T * NSW)
    out = _pooled_lookup(table, idx_flat)
    return out.reshape(B, S, EMB)


# speedup vs baseline: 2.2818x; 1.0319x over previous
import functools

import jax
import jax.numpy as jnp
from jax import lax
from jax.experimental import pallas as pl
from jax.experimental.pallas import tpu as pltpu
from jax.experimental.pallas import tpu_sc as plsc

B, S, NSW = 1024, 50, 20
EMB = 64
ROWW = 128
VPAD = 1000008
T = B * S
NC, NS = 2, 16
NW = NC * NS
TPW = T // NW
IPW = TPW * NSW
CHUNK_TOK = 16
ROWS_PER_CHUNK = CHUNK_TOK * NSW
GATHER_N = 4
GATHER_IDX = ROWS_PER_CHUNK // GATHER_N
NCHUNK = TPW // CHUNK_TOK


def _sc_kernel(table_hbm, idx_hbm, out_hbm,
               idx_v, rows_v0, rows_v1, out_v, sem0, sem1):
    wid = lax.axis_index("s") * NC + lax.axis_index("c")
    out_base = wid * (TPW * EMB)

    pltpu.sync_copy(idx_hbm.at[pl.ds(wid * IPW, IPW)], idx_v)

    def fire(g, rows_v, sem):
        for j in range(GATHER_N):
            o = g * ROWS_PER_CHUNK + j * GATHER_IDX
            pltpu.async_copy(
                table_hbm.at[idx_v.at[pl.ds(o, GATHER_IDX)]],
                rows_v.at[pl.ds(j * GATHER_IDX, GATHER_IDX)],
                sem,
            )

    def drain(rows_v, sem):
        pltpu.make_async_copy(
            table_hbm.at[pl.ds(0, ROWS_PER_CHUNK)], rows_v, sem
        ).wait()

    def compute(g, rows_v):
        @pl.loop(0, CHUNK_TOK)
        def _(t):
            r0 = t * NSW
            for c in range(EMB // 16):
                lanes = pl.ds(c * 16, 16)
                acc_a = rows_v[r0, lanes] + rows_v[r0 + 1, lanes]
                acc_b = rows_v[r0 + 2, lanes] + rows_v[r0 + 3, lanes]
                for s in range(4, NSW, 2):
                    acc_a = acc_a + rows_v[r0 + s, lanes]
                    acc_b = acc_b + rows_v[r0 + s + 1, lanes]
                out_v[pl.ds(t * EMB + c * 16, 16)] = (acc_a + acc_b) * (1.0 / NSW)
        pltpu.sync_copy(
            out_v,
            out_hbm.at[pl.ds(out_base + g * (CHUNK_TOK * EMB), CHUNK_TOK * EMB)],
        )

    fire(0, rows_v0, sem0)

    @pl.loop(0, NCHUNK, step=2)
    def _(g):
        fire(g + 1, rows_v1, sem1)
        drain(rows_v0, sem0)
        compute(g, rows_v0)

        @pl.when(g + 2 < NCHUNK)
        def _():
            fire(g + 2, rows_v0, sem0)

        drain(rows_v1, sem1)
        compute(g + 1, rows_v1)


@jax.jit
def _pooled_lookup(table, idx_flat):
    mesh = plsc.VectorSubcoreMesh(core_axis_name="c", subcore_axis_name="s")
    run = pl.kernel(
        _sc_kernel,
        out_type=jax.ShapeDtypeStruct((T * EMB,), jnp.float32),
        mesh=mesh,
        compiler_params=pltpu.CompilerParams(use_tc_tiling_on_sc=True),
        scratch_types=[
            pltpu.VMEM((IPW,), jnp.int32),
            pltpu.VMEM((ROWS_PER_CHUNK, ROWW), jnp.float32),
            pltpu.VMEM((ROWS_PER_CHUNK, ROWW), jnp.float32),
            pltpu.VMEM((CHUNK_TOK * EMB,), jnp.float32),
            pltpu.SemaphoreType.DMA,
            pltpu.SemaphoreType.DMA,
        ],
    )
    tablep = jnp.pad(table, ((0, VPAD - table.shape[0]), (0, ROWW - EMB)))
    return run(tablep, idx_flat)


def kernel(inpt, table):
    idx_flat = inpt.reshape(T * NSW)
    out = _pooled_lookup(table, idx_flat)
    return out.reshape(B, S, EMB)
